# baseline (speedup 1.0000x reference)
import functools

import jax
import jax.numpy as jnp
from jax.experimental import pallas as pl
from jax.experimental.pallas import tpu as pltpu

TOPK = 2
TB = 512
BLK = 512


def _router_body(G, x_ref, embT_ref, gates_ref, pos_ref, be_ref,
                 e0_s, e1_s, carry_s, bstart_s, gstart_s):
    p = pl.program_id(0)
    b = pl.program_id(1)
    nb = pl.num_programs(1)
    E = embT_ref.shape[1]

    iota8 = jax.lax.broadcasted_iota(jnp.int32, (TB, E), 1)

    @pl.when(p == 0)
    def _phase0():
        x = x_ref[...]
        logits = jnp.dot(x, embT_ref[...], preferred_element_type=jnp.float32)
        m0 = jnp.max(logits, axis=1, keepdims=True)
        i0 = jnp.min(jnp.where(logits == m0, iota8, E), axis=1, keepdims=True)
        l2 = jnp.where(iota8 == i0, -jnp.inf, logits)
        m1 = jnp.max(l2, axis=1, keepdims=True)
        i1 = jnp.min(jnp.where(l2 == m1, iota8, E), axis=1, keepdims=True)
        e1v = jnp.exp(m1 - m0)
        s = 1.0 + e1v
        w0 = 1.0 / s
        w1 = e1v / s
        gates_ref[...] = (jnp.where(iota8 == 0, w0, 0.0)
                          + jnp.where(iota8 == 1, w1, 0.0))
        e0_s[pl.ds(b * TB, TB), :] = i0
        e1_s[pl.ds(b * TB, TB), :] = i1
        oh0 = (iota8 == i0).astype(jnp.float32)
        oh1 = (iota8 == i1).astype(jnp.float32)
        blockcnt = jnp.sum(oh0 + oh1, axis=0, keepdims=True).astype(jnp.int32)

        @pl.when(b == 0)
        def _():
            carry_s[...] = jnp.zeros_like(carry_s)

        bstart_s[pl.ds(b, 1), :] = carry_s[...]
        carry_s[...] += blockcnt

        @pl.when(b == nb - 1)
        def _finalize():
            c = carry_s[...]
            pc = ((c + (BLK - 1)) // BLK) * BLK
            gstart = jnp.zeros_like(pc)
            for e in range(E):
                pce = pc[0:1, e:e + 1]
                lane = jax.lax.broadcasted_iota(jnp.int32, (1, E), 1)
                gstart = gstart + jnp.where(lane > e, pce, 0)
            gstart_s[...] = gstart
            pend = gstart + pc
            total = jnp.sum(pc, axis=1, keepdims=True)
            lane128 = jax.lax.broadcasted_iota(jnp.int32, be_ref.shape, 1)
            acc = jnp.zeros(be_ref.shape, jnp.int32)
            for e in range(E):
                acc = acc + (pend[0:1, e:e + 1] <= lane128 * BLK).astype(jnp.int32)
            be = jnp.minimum(acc, E - 1)
            be_ref[...] = jnp.where(lane128 == G, total, be)

    @pl.when(p == 1)
    def _phase1():
        e0 = e0_s[pl.ds(b * TB, TB), :]
        e1 = e1_s[pl.ds(b * TB, TB), :]
        oh0 = (iota8 == e0).astype(jnp.float32)
        oh1 = (iota8 == e1).astype(jnp.float32)
        r = jax.lax.broadcasted_iota(jnp.int32, (TB, TB), 0)
        cidx = jax.lax.broadcasted_iota(jnp.int32, (TB, TB), 1)
        M = (r > cidx).astype(jnp.float32)
        r0 = jnp.dot(M, oh0, preferred_element_type=jnp.float32)
        r1 = jnp.dot(M, oh1, preferred_element_type=jnp.float32)
        tot0 = jnp.sum(oh0, axis=0, keepdims=True)
        base = (gstart_s[...] + bstart_s[pl.ds(b, 1), :]).astype(jnp.float32)
        pos0 = jnp.sum((base + r0) * oh0, axis=1, keepdims=True)
        pos1 = jnp.sum((base + tot0 + r1) * oh1, axis=1, keepdims=True)
        pos_ref[...] = (jnp.where(iota8 == 0, pos0.astype(jnp.int32), 0)
                        + jnp.where(iota8 == 1, pos1.astype(jnp.int32), 0))


def _router(x2, embT, N, D, E, G):
    nb = N // TB
    return pl.pallas_call(
        functools.partial(_router_body, G),
        grid=(2, nb),
        in_specs=[
            pl.BlockSpec((TB, D), lambda p, b: (b, 0)),
            pl.BlockSpec((D, E), lambda p, b: (0, 0)),
        ],
        out_specs=[
            pl.BlockSpec((TB, E), lambda p, b: (b + p * (nb - b), 0)),
            pl.BlockSpec((TB, E), lambda p, b: (b + (1 - p) * (nb - b), 0)),
            pl.BlockSpec((8, 128), lambda p, b: (0, 0)),
        ],
        out_shape=[
            jax.ShapeDtypeStruct((N + TB, E), jnp.float32),
            jax.ShapeDtypeStruct((N + TB, E), jnp.int32),
            jax.ShapeDtypeStruct((8, 128), jnp.int32),
        ],
        scratch_shapes=[
            pltpu.VMEM((N, 1), jnp.int32),
            pltpu.VMEM((N, 1), jnp.int32),
            pltpu.VMEM((1, E), jnp.int32),
            pltpu.VMEM((nb, E), jnp.int32),
            pltpu.VMEM((1, E), jnp.int32),
        ],
        compiler_params=pltpu.CompilerParams(
            dimension_semantics=("arbitrary", "arbitrary"),
        ),
    )(x2, embT)


def _mlp_body(be_ref, xs_ref, W1_ref, b1_ref, W2_ref, b2_ref, gp_ref, out_ref):
    g = pl.program_id(0)
    G = pl.num_programs(0)
    total = be_ref[G]

    @pl.when(g * BLK < total)
    def _():
        xv = xs_ref[...]
        h = jnp.dot(xv, W1_ref[0], preferred_element_type=jnp.float32) + b1_ref[0]
        h = jnp.maximum(h, 0.0)
        y = jnp.dot(h, W2_ref[0], preferred_element_type=jnp.float32) + b2_ref[0]
        out_ref[...] = y * gp_ref[...]


def _grouped_mlp(xs, W1, b1r, W2, b2r, gatep2, be_arr, P, D, F, G):
    grid_spec = pltpu.PrefetchScalarGridSpec(
        num_scalar_prefetch=1,
        grid=(G,),
        in_specs=[
            pl.BlockSpec((BLK, D), lambda g, be: (g, 0)),
            pl.BlockSpec((1, D, F), lambda g, be: (be[g], 0, 0)),
            pl.BlockSpec((1, 1, F), lambda g, be: (be[g], 0, 0)),
            pl.BlockSpec((1, F, D), lambda g, be: (be[g], 0, 0)),
            pl.BlockSpec((1, 1, D), lambda g, be: (be[g], 0, 0)),
            pl.BlockSpec((BLK, 1), lambda g, be: (g, 0)),
        ],
        out_specs=pl.BlockSpec((BLK, D), lambda g, be: (g, 0)),
    )
    return pl.pallas_call(
        _mlp_body,
        grid_spec=grid_spec,
        out_shape=jax.ShapeDtypeStruct((P, D), jnp.float32),
        compiler_params=pltpu.CompilerParams(
            dimension_semantics=("arbitrary",),
        ),
    )(be_arr, xs, W1, b1r, W2, b2r, gatep2)


def kernel(x, expert_embeddings, W1, b1, W2, b2):
    B, S, D = x.shape
    E, _, F = W1.shape
    N = B * S
    NS = N * TOPK
    P = NS + E * BLK
    G = P // BLK

    x2 = x.reshape(N, D)
    embT = expert_embeddings.T
    b1r = b1.reshape(E, 1, F)
    b2r = b2.reshape(E, 1, D)

    gates8, pos8, be_out = _router(x2, embT, N, D, E, G)

    gates8 = gates8[:N]
    pos8 = pos8[:N]
    posf = pos8[:, :TOPK].reshape(NS)
    gatesf = gates8[:, :TOPK].reshape(NS)
    be_arr = be_out[0, :G + 1]

    tokf = jnp.arange(NS, dtype=jnp.int32) // TOPK
    invtok = jnp.zeros((P,), jnp.int32).at[posf].set(tokf)
    gatep = jnp.zeros((P,), jnp.float32).at[posf].set(gatesf)
    xs = jnp.take(x2, invtok, axis=0)

    ys = _grouped_mlp(xs, W1, b1r, W2, b2r, gatep.reshape(P, 1), be_arr,
                      P, D, F, G)

    out2 = jnp.take(ys, pos8[:, 0], axis=0) + jnp.take(ys, pos8[:, 1], axis=0)
    return out2.reshape(B, S, D)
